# baseline (speedup 1.0000x reference)
import functools

import jax
import jax.numpy as jnp
from jax import lax
from jax.experimental import pallas as pl
from jax.experimental.pallas import tpu as pltpu
from jax.experimental.pallas import tpu_sc as plsc

_NC = 2
_NS = 16

_V = 1000000
_D = 64
_B = 4096
_S = 200

_RT = 7813
_TR = 488
_TR_LAST = _RT - 15 * _TR
_GRP = 10


@functools.lru_cache(maxsize=None)
def _build():
    d_per_core = _D // _NC
    n_pairs = _S // (2 * _GRP)
    mesh = plsc.VectorSubcoreMesh(core_axis_name="c", subcore_axis_name="s")

    @functools.partial(
        pl.kernel,
        out_type=jax.ShapeDtypeStruct((_S, _D, _B), jnp.float32),
        mesh=mesh,
        scratch_types=[
            pltpu.VMEM((_S, 128), jnp.int32),
            pltpu.VMEM((_S, 128), jnp.int32),
            pltpu.VMEM((2, _GRP, 128), jnp.float32),
            pltpu.VMEM((2, _GRP, 128), jnp.float32),
            pltpu.VMEM_SHARED((_RT * 128,), jnp.float32),
            pltpu.SemaphoreType.DMA,
            pltpu.SemaphoreType.DMA,
            pltpu.SemaphoreType.DMA,
            pltpu.SemaphoreType.DMA,
        ],
    )
    def k(xs_hbm, emb_hbm, out_hbm, idx_a, idx_b, ob_a, ob_b,
          row, sem_idx, sem_row, sem_g, sem_w):
        c = lax.axis_index("c")
        t = lax.axis_index("s")
        b0 = t * 256

        ia = pltpu.make_async_copy(xs_hbm.at[:, pl.ds(b0, 128)], idx_a, sem_idx)
        ib = pltpu.make_async_copy(xs_hbm.at[:, pl.ds(b0 + 128, 128)], idx_b,
                                   sem_idx)
        ia.start()
        ib.start()
        ia.wait()
        ib.wait()

        def stage_rows(dbase, r0, n):
            cp = pltpu.make_async_copy(
                emb_hbm.at[pl.ds((dbase + r0) * 128, n * 128)],
                row.at[pl.ds(r0 * 128, n * 128)], sem_row)
            cp.start()
            cp.wait()

        def dbody(di, carry):
            d = c * d_per_core + di
            dbase = d * _RT

            @pl.when(t < _NS - 1)
            def _():
                stage_rows(dbase, t * _TR, _TR)

            @pl.when(t == _NS - 1)
            def _():
                stage_rows(dbase, 15 * _TR, _TR_LAST)

            plsc.subcore_barrier()

            def make_wb(par, s0):
                wa = pltpu.make_async_copy(
                    ob_a.at[par],
                    out_hbm.at[pl.ds(s0, _GRP), d, pl.ds(b0, 128)], sem_w)
                wb = pltpu.make_async_copy(
                    ob_b.at[par],
                    out_hbm.at[pl.ds(s0, _GRP), d, pl.ds(b0 + 128, 128)],
                    sem_w)
                return wa, wb

            def pbody(p, cc):
                for par in range(2):
                    s0 = (p * 2 + par) * _GRP

                    @pl.when(p > 0)
                    def _(par=par, s0=s0):
                        wa, wb = make_wb(par, s0 - 2 * _GRP)
                        wa.wait()
                        wb.wait()

                    copies = []
                    for j in range(_GRP):
                        c1 = pltpu.make_async_copy(
                            row.at[idx_a.at[s0 + j]], ob_a.at[par, j], sem_g)
                        c2 = pltpu.make_async_copy(
                            row.at[idx_b.at[s0 + j]], ob_b.at[par, j], sem_g)
                        c1.start()
                        c2.start()
                        copies.append(c1)
                        copies.append(c2)
                    for cp in copies:
                        cp.wait()
                    wa, wb = make_wb(par, s0)
                    wa.start()
                    wb.start()
                return cc

            lax.fori_loop(0, n_pairs, pbody, 0)
            for par in range(2):
                wa, wb = make_wb(par, (n_pairs * 2 - 2 + par) * _GRP)
                wa.wait()
                wb.wait()
            plsc.subcore_barrier()
            return carry

        lax.fori_loop(0, d_per_core, dbody, 0)

    return k


def kernel(xs, emb):
    assert xs.shape == (_B, _S) and emb.shape == (_V, _D)
    emb_p = jnp.pad((emb + jnp.float32(1e-30)).T, ((0, 0), (0, 64)))
    out3 = _build()(xs.T, emb_p.reshape(_D * _RT * 128))
    return out3.transpose(2, 0, 1)


# speedup vs baseline: 4.3684x; 4.3684x over previous
import functools

import jax
import jax.numpy as jnp
from jax import lax
from jax.experimental import pallas as pl
from jax.experimental.pallas import tpu as pltpu
from jax.experimental.pallas import tpu_sc as plsc

_NC = 2
_NS = 16

_V = 1000000
_D = 64
_B = 4096
_S = 200

_RT = 7816
_TR = 488
_TR_LAST = _RT - 15 * _TR
_GRP = 10


@functools.lru_cache(maxsize=None)
def _build():
    d_per_core = _D // _NC
    n_pairs = _S // (2 * _GRP)
    mesh = plsc.VectorSubcoreMesh(core_axis_name="c", subcore_axis_name="s")

    @functools.partial(
        pl.kernel,
        out_type=jax.ShapeDtypeStruct((_S, _D, _B), jnp.float32),
        mesh=mesh,
        scratch_types=[
            pltpu.VMEM((_S, 1, 128), jnp.int32),
            pltpu.VMEM((_S, 1, 128), jnp.int32),
            pltpu.VMEM((_GRP, 128), jnp.float32),
            pltpu.VMEM((_GRP, 128), jnp.float32),
            pltpu.VMEM((_GRP, 128), jnp.float32),
            pltpu.VMEM((_GRP, 128), jnp.float32),
            pltpu.VMEM_SHARED((1, _RT * 128), jnp.float32),
            pltpu.SemaphoreType.DMA,
            pltpu.SemaphoreType.DMA,
            pltpu.SemaphoreType.DMA,
            pltpu.SemaphoreType.DMA,
        ],
    )
    def k(xs_hbm, emb_hbm, out_hbm, idx_a, idx_b, ob00, ob01, ob10, ob11,
          row, sem_idx, sem_row, sem_g, sem_w):
        obs = ((ob00, ob01), (ob10, ob11))
        c = lax.axis_index("c")
        t = lax.axis_index("s")
        b0 = t * 256

        ia = pltpu.make_async_copy(xs_hbm.at[:, pl.ds(b0, 128)], idx_a.at[:, 0, :], sem_idx)
        ib = pltpu.make_async_copy(xs_hbm.at[:, pl.ds(b0 + 128, 128)],
                                   idx_b.at[:, 0, :], sem_idx)
        ia.start()
        ib.start()
        ia.wait()
        ib.wait()


        def stage_rows(dbase, r0, n):
            cp = pltpu.make_async_copy(
                emb_hbm.at[:, pl.ds((dbase + r0) * 128, n * 128)],
                row.at[:, pl.ds(r0 * 128, n * 128)], sem_row)
            cp.start()
            cp.wait()

        def dbody(di, carry):
            d = c * d_per_core + di
            dbase = d * _RT

            @pl.when(t < _NS - 1)
            def _():
                stage_rows(dbase, t * _TR, _TR)

            @pl.when(t == _NS - 1)
            def _():
                stage_rows(dbase, 15 * _TR, _TR_LAST)

            plsc.subcore_barrier()

            def make_wb(par, s0):
                wa = pltpu.make_async_copy(
                    obs[par][0],
                    out_hbm.at[pl.ds(s0, _GRP), d, pl.ds(b0, 128)], sem_w)
                wb = pltpu.make_async_copy(
                    obs[par][1],
                    out_hbm.at[pl.ds(s0, _GRP), d, pl.ds(b0 + 128, 128)],
                    sem_w)
                return wa, wb

            def pbody(p, cc):
                for par in range(2):
                    s0 = (p * 2 + par) * _GRP

                    @pl.when(p > 0)
                    def _(par=par, s0=s0):
                        wa, wb = make_wb(par, s0 - 2 * _GRP)
                        wa.wait()
                        wb.wait()

                    copies = []
                    for j in range(_GRP):
                        c1 = pltpu.make_async_copy(
                            row.at[idx_a.at[s0 + j]], obs[par][0].at[pl.ds(j, 1), :], sem_g)
                        c2 = pltpu.make_async_copy(
                            row.at[idx_b.at[s0 + j]], obs[par][1].at[pl.ds(j, 1), :], sem_g)
                        c1.start()
                        c2.start()
                        copies.append(c1)
                        copies.append(c2)
                    for cp in copies:
                        cp.wait()
                    wa, wb = make_wb(par, s0)
                    wa.start()
                    wb.start()
                return cc

            lax.fori_loop(0, n_pairs, pbody, 0)
            for par in range(2):
                wa, wb = make_wb(par, (n_pairs * 2 - 2 + par) * _GRP)
                wa.wait()
                wb.wait()
            plsc.subcore_barrier()
            return carry

        lax.fori_loop(0, d_per_core, dbody, 0)

    return k


def kernel(xs, emb):
    assert xs.shape == (_B, _S) and emb.shape == (_V, _D)
    emb_p = jnp.pad((emb + jnp.float32(1e-30)).T, ((0, 0), (0, _RT * 128 - _V)))
    out3 = _build()(xs.T, emb_p.reshape(1, _D * _RT * 128))
    return out3.transpose(2, 0, 1)


# speedup vs baseline: 4.4592x; 1.0208x over previous
import functools

import jax
import jax.numpy as jnp
from jax import lax
from jax.experimental import pallas as pl
from jax.experimental.pallas import tpu as pltpu
from jax.experimental.pallas import tpu_sc as plsc

_NC = 2
_NS = 16

_V = 1000000
_D = 64
_B = 4096
_S = 200

_RT = 7816
_TR = 488
_TR_LAST = _RT - 15 * _TR
_GRP = 10


@functools.lru_cache(maxsize=None)
def _build():
    d_per_core = _D // _NC
    n_pairs = _S // (2 * _GRP)
    mesh = plsc.VectorSubcoreMesh(core_axis_name="c", subcore_axis_name="s")

    @functools.partial(
        pl.kernel,
        out_type=jax.ShapeDtypeStruct((_S, _D, _B), jnp.float32),
        mesh=mesh,
        scratch_types=[
            pltpu.VMEM((_S, 1, 128), jnp.int32),
            pltpu.VMEM((_S, 1, 128), jnp.int32),
            pltpu.VMEM((_GRP, 128), jnp.float32),
            pltpu.VMEM((_GRP, 128), jnp.float32),
            pltpu.VMEM((_GRP, 128), jnp.float32),
            pltpu.VMEM((_GRP, 128), jnp.float32),
            pltpu.VMEM_SHARED((1, _RT * 128), jnp.float32),
            pltpu.SemaphoreType.DMA,
            pltpu.SemaphoreType.DMA,
            pltpu.SemaphoreType.DMA,
            pltpu.SemaphoreType.DMA,
        ],
    )
    def k(xs_hbm, emb_hbm, out_hbm, idx_a, idx_b, ob00, ob01, ob10, ob11,
          row, sem_idx, sem_row, sem_g, sem_w):
        obs = ((ob00, ob01), (ob10, ob11))
        c = lax.axis_index("c")
        t = lax.axis_index("s")
        b0 = t * 256

        ia = pltpu.make_async_copy(xs_hbm.at[:, pl.ds(b0, 128)], idx_a.at[:, 0, :], sem_idx)
        ib = pltpu.make_async_copy(xs_hbm.at[:, pl.ds(b0 + 128, 128)],
                                   idx_b.at[:, 0, :], sem_idx)
        ia.start()
        ib.start()
        ia.wait()
        ib.wait()


        def stage_rows(dbase, r0, n):
            cp = pltpu.make_async_copy(
                emb_hbm.at[:, pl.ds((dbase + r0) * 128, n * 128)],
                row.at[:, pl.ds(r0 * 128, n * 128)], sem_row)
            cp.start()
            cp.wait()

        def dbody(di, carry):
            d = c * d_per_core + di
            dbase = d * _RT

            @pl.when(t < _NS - 1)
            def _():
                stage_rows(dbase, t * _TR, _TR)

            @pl.when(t == _NS - 1)
            def _():
                stage_rows(dbase, 15 * _TR, _TR_LAST)

            plsc.subcore_barrier()

            def make_wb(par, s0):
                wa = pltpu.make_async_copy(
                    obs[par][0],
                    out_hbm.at[pl.ds(s0, _GRP), d, pl.ds(b0, 128)], sem_w)
                wb = pltpu.make_async_copy(
                    obs[par][1],
                    out_hbm.at[pl.ds(s0, _GRP), d, pl.ds(b0 + 128, 128)],
                    sem_w)
                return wa, wb

            def fire_gathers(par, s0):
                for j in range(_GRP):
                    pltpu.make_async_copy(
                        row.at[idx_a.at[s0 + j]],
                        obs[par][0].at[pl.ds(j, 1), :], sem_g).start()
                    pltpu.make_async_copy(
                        row.at[idx_b.at[s0 + j]],
                        obs[par][1].at[pl.ds(j, 1), :], sem_g).start()

            def drain_gathers(par):
                for j in range(_GRP):
                    pltpu.make_async_copy(
                        row.at[idx_a.at[0]],
                        obs[par][0].at[pl.ds(j, 1), :], sem_g).wait()
                    pltpu.make_async_copy(
                        row.at[idx_b.at[0]],
                        obs[par][1].at[pl.ds(j, 1), :], sem_g).wait()

            def pbody(p, cc):
                for par in range(2):
                    s0 = (p * 2 + par) * _GRP

                    @pl.when(p > 0)
                    def _(par=par, s0=s0):
                        wa, wb = make_wb(par, s0 - 2 * _GRP)
                        wa.wait()
                        wb.wait()

                    fire_gathers(par, s0)
                    if par == 1:
                        drain_gathers(0)
                        wa, wb = make_wb(0, s0 - _GRP)
                        wa.start()
                        wb.start()
                    else:
                        @pl.when(p > 0)
                        def _(s0=s0):
                            drain_gathers(1)
                            wa, wb = make_wb(1, s0 - _GRP)
                            wa.start()
                            wb.start()
                return cc

            lax.fori_loop(0, n_pairs, pbody, 0)
            drain_gathers(1)
            wlast = make_wb(1, (n_pairs * 2 - 1) * _GRP)
            wlast[0].start()
            wlast[1].start()
            for par in range(2):
                wa, wb = make_wb(par, (n_pairs * 2 - 2 + par) * _GRP)
                wa.wait()
                wb.wait()
            plsc.subcore_barrier()
            return carry

        lax.fori_loop(0, d_per_core, dbody, 0)

    return k


def kernel(xs, emb):
    assert xs.shape == (_B, _S) and emb.shape == (_V, _D)
    emb_p = jnp.pad((emb + jnp.float32(1e-30)).T, ((0, 0), (0, _RT * 128 - _V)))
    out3 = _build()(xs.T, emb_p.reshape(1, _D * _RT * 128))
    return out3.transpose(2, 0, 1)


# speedup vs baseline: 4.6421x; 1.0410x over previous
import functools

import jax
import jax.numpy as jnp
from jax import lax
from jax.experimental import pallas as pl
from jax.experimental.pallas import tpu as pltpu
from jax.experimental.pallas import tpu_sc as plsc

_NC = 2
_NS = 16

_V = 1000000
_D = 64
_B = 4096
_S = 200

_RT = 7816
_TR = 488
_TR_LAST = _RT - 15 * _TR
_GRP = 20


@functools.lru_cache(maxsize=None)
def _build():
    d_per_core = _D // _NC
    n_pairs = _S // (2 * _GRP)
    mesh = plsc.VectorSubcoreMesh(core_axis_name="c", subcore_axis_name="s")

    @functools.partial(
        pl.kernel,
        out_type=jax.ShapeDtypeStruct((_S, _D, _B), jnp.float32),
        mesh=mesh,
        scratch_types=[
            pltpu.VMEM((_S, 1, 128), jnp.int32),
            pltpu.VMEM((_S, 1, 128), jnp.int32),
            pltpu.VMEM((_GRP, 128), jnp.float32),
            pltpu.VMEM((_GRP, 128), jnp.float32),
            pltpu.VMEM((_GRP, 128), jnp.float32),
            pltpu.VMEM((_GRP, 128), jnp.float32),
            pltpu.VMEM_SHARED((1, _RT * 128), jnp.float32),
            pltpu.SemaphoreType.DMA,
            pltpu.SemaphoreType.DMA,
            pltpu.SemaphoreType.DMA,
            pltpu.SemaphoreType.DMA,
        ],
    )
    def k(xs_hbm, emb_hbm, out_hbm, idx_a, idx_b, ob00, ob01, ob10, ob11,
          row, sem_idx, sem_row, sem_g, sem_w):
        obs = ((ob00, ob01), (ob10, ob11))
        c = lax.axis_index("c")
        t = lax.axis_index("s")
        b0 = t * 256

        ia = pltpu.make_async_copy(xs_hbm.at[:, pl.ds(b0, 128)], idx_a.at[:, 0, :], sem_idx)
        ib = pltpu.make_async_copy(xs_hbm.at[:, pl.ds(b0 + 128, 128)],
                                   idx_b.at[:, 0, :], sem_idx)
        ia.start()
        ib.start()
        ia.wait()
        ib.wait()


        def stage_rows(dbase, r0, n):
            cp = pltpu.make_async_copy(
                emb_hbm.at[:, pl.ds((dbase + r0) * 128, n * 128)],
                row.at[:, pl.ds(r0 * 128, n * 128)], sem_row)
            cp.start()
            cp.wait()

        def dbody(di, carry):
            d = c * d_per_core + di
            dbase = d * _RT

            @pl.when(t < _NS - 1)
            def _():
                stage_rows(dbase, t * _TR, _TR)

            @pl.when(t == _NS - 1)
            def _():
                stage_rows(dbase, 15 * _TR, _TR_LAST)

            plsc.subcore_barrier()

            def make_wb(par, s0):
                wa = pltpu.make_async_copy(
                    obs[par][0],
                    out_hbm.at[pl.ds(s0, _GRP), d, pl.ds(b0, 128)], sem_w)
                wb = pltpu.make_async_copy(
                    obs[par][1],
                    out_hbm.at[pl.ds(s0, _GRP), d, pl.ds(b0 + 128, 128)],
                    sem_w)
                return wa, wb

            def fire_gathers(par, s0):
                for j in range(_GRP):
                    pltpu.make_async_copy(
                        row.at[idx_a.at[s0 + j]],
                        obs[par][0].at[pl.ds(j, 1), :], sem_g).start()
                    pltpu.make_async_copy(
                        row.at[idx_b.at[s0 + j]],
                        obs[par][1].at[pl.ds(j, 1), :], sem_g).start()

            def drain_gathers(par):
                for j in range(_GRP):
                    pltpu.make_async_copy(
                        row.at[idx_a.at[0]],
                        obs[par][0].at[pl.ds(j, 1), :], sem_g).wait()
                    pltpu.make_async_copy(
                        row.at[idx_b.at[0]],
                        obs[par][1].at[pl.ds(j, 1), :], sem_g).wait()

            def pbody(p, cc):
                for par in range(2):
                    s0 = (p * 2 + par) * _GRP

                    @pl.when(p > 0)
                    def _(par=par, s0=s0):
                        wa, wb = make_wb(par, s0 - 2 * _GRP)
                        wa.wait()
                        wb.wait()

                    fire_gathers(par, s0)
                    if par == 1:
                        drain_gathers(0)
                        wa, wb = make_wb(0, s0 - _GRP)
                        wa.start()
                        wb.start()
                    else:
                        @pl.when(p > 0)
                        def _(s0=s0):
                            drain_gathers(1)
                            wa, wb = make_wb(1, s0 - _GRP)
                            wa.start()
                            wb.start()
                return cc

            lax.fori_loop(0, n_pairs, pbody, 0)
            drain_gathers(1)
            wlast = make_wb(1, (n_pairs * 2 - 1) * _GRP)
            wlast[0].start()
            wlast[1].start()
            for par in range(2):
                wa, wb = make_wb(par, (n_pairs * 2 - 2 + par) * _GRP)
                wa.wait()
                wb.wait()
            plsc.subcore_barrier()
            return carry

        lax.fori_loop(0, d_per_core, dbody, 0)

    return k


def kernel(xs, emb):
    assert xs.shape == (_B, _S) and emb.shape == (_V, _D)
    emb_p = jnp.pad((emb + jnp.float32(1e-30)).T, ((0, 0), (0, _RT * 128 - _V)))
    out3 = _build()(xs.T, emb_p.reshape(1, _D * _RT * 128))
    return out3.transpose(2, 0, 1)
